# Initial kernel scaffold; baseline (speedup 1.0000x reference)
import jax
import jax.numpy as jnp
from jax.experimental import pallas as pl


def kernel(x, edge_index, params):
    raise NotImplementedError("write your pallas kernel here")


# speedup vs baseline: 2.5184x; 2.5184x over previous
import functools

import jax
import jax.numpy as jnp
from jax import lax
from jax.experimental import pallas as pl
from jax.experimental.pallas import tpu as pltpu
from jax.experimental.pallas import tpu_sc as plsc

N_NODES = 10000
N_PAD = 10240
D_FEAT = 128
HID = 256
EPS = 1e-3

BLK = 1024
GRID = N_PAD // BLK

CHUNK = 128
N_TILES = 16
EPW_ROWS = 160
STAGE_ROWS = 80
E_PAD = N_TILES * EPW_ROWS * CHUNK
ROWS_PER_TILE = N_PAD // N_TILES


def _fold(p):
    s = p["gamma"] / jnp.sqrt(p["var"] + EPS)
    return p["W"] * s[None, :], ((p["b"] - p["mean"]) * s + p["beta"])[None, :], p["alpha"][None, :]


def _make_dense(widths, dout, split, colsum):
    nparts = len(widths)

    def body(*refs):
        xs = refs[:nparts]
        ws = refs[nparts:2 * nparts]
        b_ref = refs[2 * nparts]
        a_ref = refs[2 * nparts + 1]
        outs = refs[2 * nparts + 2:]
        acc = jnp.dot(xs[0][...], ws[0][...], preferred_element_type=jnp.float32)
        for p in range(1, nparts):
            acc += jnp.dot(xs[p][...], ws[p][...], preferred_element_type=jnp.float32)
        y = acc + b_ref[...]
        z = jnp.where(y > 0.0, y, y * a_ref[...])
        if split:
            outs[0][...] = z[:, :128]
            outs[1][...] = z[:, 128:]
            k = 2
        else:
            outs[0][...] = z
            k = 1
        if colsum:
            cs_ref = outs[k]
            i = pl.program_id(0)

            @pl.when(i == 0)
            def _():
                cs_ref[...] = jnp.zeros_like(cs_ref)

            rows = i * BLK + lax.broadcasted_iota(jnp.int32, (BLK, 1), 0)
            m = (rows < N_NODES).astype(jnp.float32)
            off = 0
            for p in range(nparts):
                w = widths[p]
                cs_ref[:, off:off + w] += jnp.sum(xs[p][...] * m, axis=0, keepdims=True)
                off += w

    sum_w = sum(widths)
    in_specs = (
        [pl.BlockSpec((BLK, w), lambda i: (i, 0)) for w in widths]
        + [pl.BlockSpec((w, dout), lambda i: (0, 0)) for w in widths]
        + [pl.BlockSpec((1, dout), lambda i: (0, 0))] * 2
    )
    if split:
        out_shape = [jax.ShapeDtypeStruct((N_PAD, 128), jnp.float32)] * 2
        out_specs = [pl.BlockSpec((BLK, 128), lambda i: (i, 0))] * 2
    else:
        out_shape = [jax.ShapeDtypeStruct((N_PAD, dout), jnp.float32)]
        out_specs = [pl.BlockSpec((BLK, dout), lambda i: (i, 0))]
    if colsum:
        out_shape = out_shape + [jax.ShapeDtypeStruct((1, sum_w), jnp.float32)]
        out_specs = out_specs + [pl.BlockSpec((1, sum_w), lambda i: (0, 0))]

    return pl.pallas_call(
        body,
        grid=(GRID,),
        in_specs=in_specs,
        out_specs=out_specs,
        out_shape=out_shape,
    )


def _dense(parts, W, b, a, row_splits, split_out, colsum=False):
    widths = [int(p.shape[1]) for p in parts]
    ws = [W[s:s + w] for s, w in zip(row_splits, widths)]
    fn = _make_dense(tuple(widths), int(W.shape[1]), split_out, colsum)
    return fn(*parts, *ws, b, a)


def _colsum_pair_body(a0_ref, a1_ref, o0_ref, o1_ref):
    i = pl.program_id(0)

    @pl.when(i == 0)
    def _():
        o0_ref[...] = jnp.zeros_like(o0_ref)
        o1_ref[...] = jnp.zeros_like(o1_ref)

    rows = i * BLK + lax.broadcasted_iota(jnp.int32, (BLK, 1), 0)
    m = (rows < N_NODES).astype(jnp.float32)
    o0_ref[...] += jnp.sum(a0_ref[...] * m, axis=0, keepdims=True)
    o1_ref[...] += jnp.sum(a1_ref[...] * m, axis=0, keepdims=True)


_colsum_pair = pl.pallas_call(
    _colsum_pair_body,
    grid=(GRID,),
    in_specs=[pl.BlockSpec((BLK, 128), lambda i: (i, 0))] * 2,
    out_specs=[pl.BlockSpec((1, 128), lambda i: (0, 0))] * 2,
    out_shape=[jax.ShapeDtypeStruct((1, 128), jnp.float32)] * 2,
)


def _post_body(p_ref, w1_ref, b1_ref, a1_ref, w2t_ref, b2_ref, o_ref):
    y = jnp.dot(p_ref[...], w1_ref[...], preferred_element_type=jnp.float32) + b1_ref[...]
    y = jnp.where(y > 0.0, y, y * a1_ref[...])
    o_ref[...] = jnp.sum(y * w2t_ref[...], axis=1, keepdims=True) + b2_ref[...]


_post = pl.pallas_call(
    _post_body,
    out_shape=jax.ShapeDtypeStruct((1, 1), jnp.float32),
)


def _sc_agg_body(z0, z1, src_hbm, dst_hbm, out0, out1,
                 src_v, dst_v, rows_v, agg_sh, sem):
    c = lax.axis_index("c")
    s = lax.axis_index("s")

    zero16 = jnp.zeros((16,), jnp.float32)

    def zb(i, carry):
        for k in range(8):
            rows_v[i, pl.ds(k * 16, 16)] = zero16
        return carry

    lax.fori_loop(0, CHUNK, zb, 0)

    zbase = s * ROWS_PER_TILE
    for r0 in range(0, ROWS_PER_TILE, CHUNK):
        pltpu.sync_copy(rows_v, agg_sh.at[pl.ds(zbase + r0, CHUNK)])
    plsc.subcore_barrier()

    def run(zt):
        for st in range(EPW_ROWS // STAGE_ROWS):
            pltpu.sync_copy(
                src_hbm.at[pl.ds(s * EPW_ROWS + st * STAGE_ROWS, STAGE_ROWS)], src_v)
            pltpu.sync_copy(
                dst_hbm.at[pl.ds(s * EPW_ROWS + st * STAGE_ROWS, STAGE_ROWS)], dst_v)

            def cb(j, carry):
                pltpu.async_copy(zt.at[src_v.at[j]], rows_v, sem).wait()
                pltpu.sync_copy(rows_v, agg_sh.at[dst_v.at[j]], add=True)
                return carry
            lax.fori_loop(0, STAGE_ROWS, cb, 0)

    @pl.when(c == 0)
    def _():
        run(z0)

    @pl.when(c == 1)
    def _():
        run(z1)

    plsc.subcore_barrier()

    @pl.when(c == 0)
    def _():
        pltpu.sync_copy(agg_sh.at[pl.ds(zbase, ROWS_PER_TILE)],
                        out0.at[pl.ds(zbase, ROWS_PER_TILE)])

    @pl.when(c == 1)
    def _():
        pltpu.sync_copy(agg_sh.at[pl.ds(zbase, ROWS_PER_TILE)],
                        out1.at[pl.ds(zbase, ROWS_PER_TILE)])


def _make_sc_agg():
    return pl.kernel(
        _sc_agg_body,
        out_type=[jax.ShapeDtypeStruct((N_PAD, 128), jnp.float32)] * 2,
        mesh=plsc.VectorSubcoreMesh(core_axis_name="c", subcore_axis_name="s"),
        scratch_types=[
            pltpu.VMEM((STAGE_ROWS, CHUNK), jnp.int32),
            pltpu.VMEM((STAGE_ROWS, CHUNK), jnp.int32),
            pltpu.VMEM((CHUNK, 128), jnp.float32),
            pltpu.VMEM_SHARED((N_PAD, 128), jnp.float32),
            pltpu.SemaphoreType.DMA,
        ],
    )


def kernel(x, edge_index, params):
    W1, b1, a1 = _fold(params["pre1"])
    W2, b2, a2 = _fold(params["pre2"])
    Wc = [_fold(params["conv%d" % i]) for i in (1, 2, 3, 4)]
    Wp1, bp1, ap1 = _fold(params["post1"])
    Wp2, bp2, _ = _fold(params["post2"])

    xp = jnp.pad(x, ((0, N_PAD - N_NODES), (0, 0)))
    src = edge_index[0]
    dst = edge_index[1]
    pad_e = E_PAD - src.shape[0]
    srcp = jnp.concatenate([src, jnp.zeros((pad_e,), jnp.int32)])
    dstp = jnp.concatenate([dst, jnp.full((pad_e,), N_NODES, jnp.int32)])
    src2d = srcp.reshape(E_PAD // CHUNK, CHUNK)
    dst2d = dstp.reshape(E_PAD // CHUNK, CHUNK)

    sc_agg = _make_sc_agg()

    (t,) = _dense([xp], W1, b1, a1, [0], split_out=False)
    (h,) = _dense([t], W2, b2, a2, [0], split_out=False)

    Wk, bk, ak = Wc[0]
    z0, z1 = _dense([h], Wk, bk, ak, [0], split_out=True)
    g1_0, g1_1 = sc_agg(z0, z1, src2d, dst2d)

    Wk, bk, ak = Wc[1]
    z0, z1 = _dense([g1_0, g1_1, h], Wk, bk, ak, [0, 128, 256], split_out=True)
    g2_0, g2_1 = sc_agg(z0, z1, src2d, dst2d)

    Wk, bk, ak = Wc[2]
    z0, z1 = _dense([g2_0, g2_1, g1_0, g1_1, h], Wk, bk, ak,
                    [0, 128, 256, 384, 512], split_out=True)
    g3_0, g3_1 = sc_agg(z0, z1, src2d, dst2d)

    Wk, bk, ak = Wc[3]
    z0, z1, cs_in = _dense([g3_0, g3_1, g2_0, g2_1, g1_0, g1_1, h], Wk, bk, ak,
                           [0, 128, 256, 384, 512, 640, 768],
                           split_out=True, colsum=True)
    g4_0, g4_1 = sc_agg(z0, z1, src2d, dst2d)
    cs4_0, cs4_1 = _colsum_pair(g4_0, g4_1)

    pooled = jnp.concatenate([cs4_0, cs4_1, cs_in], axis=1)

    y = _post(pooled, Wp1, bp1, ap1, Wp2.T, bp2)
    return y.reshape((1,))


# speedup vs baseline: 2.8258x; 1.1220x over previous
import functools

import jax
import jax.numpy as jnp
from jax import lax
from jax.experimental import pallas as pl
from jax.experimental.pallas import tpu as pltpu
from jax.experimental.pallas import tpu_sc as plsc

N_NODES = 10000
N_PAD = 10240
D_FEAT = 128
HID = 256
EPS = 1e-3

BLK = 1024
GRID = N_PAD // BLK

CHUNK = 128
N_TILES = 16
EPW_ROWS = 160
STAGE_ROWS = 32
E_PAD = N_TILES * EPW_ROWS * CHUNK
ROWS_PER_TILE = N_PAD // N_TILES


def _fold(p):
    s = p["gamma"] / jnp.sqrt(p["var"] + EPS)
    return p["W"] * s[None, :], ((p["b"] - p["mean"]) * s + p["beta"])[None, :], p["alpha"][None, :]


def _make_dense(widths, dout, split, colsum):
    nparts = len(widths)

    def body(*refs):
        xs = refs[:nparts]
        ws = refs[nparts:2 * nparts]
        b_ref = refs[2 * nparts]
        a_ref = refs[2 * nparts + 1]
        outs = refs[2 * nparts + 2:]
        acc = jnp.dot(xs[0][...], ws[0][...], preferred_element_type=jnp.float32)
        for p in range(1, nparts):
            acc += jnp.dot(xs[p][...], ws[p][...], preferred_element_type=jnp.float32)
        y = acc + b_ref[...]
        z = jnp.where(y > 0.0, y, y * a_ref[...])
        if split:
            outs[0][...] = z[:, :128]
            outs[1][...] = z[:, 128:]
            k = 2
        else:
            outs[0][...] = z
            k = 1
        if colsum:
            cs_ref = outs[k]
            i = pl.program_id(0)

            @pl.when(i == 0)
            def _():
                cs_ref[...] = jnp.zeros_like(cs_ref)

            rows = i * BLK + lax.broadcasted_iota(jnp.int32, (BLK, 1), 0)
            m = (rows < N_NODES).astype(jnp.float32)
            off = 0
            for p in range(nparts):
                w = widths[p]
                cs_ref[:, off:off + w] += jnp.sum(xs[p][...] * m, axis=0, keepdims=True)
                off += w

    sum_w = sum(widths)
    in_specs = (
        [pl.BlockSpec((BLK, w), lambda i: (i, 0)) for w in widths]
        + [pl.BlockSpec((w, dout), lambda i: (0, 0)) for w in widths]
        + [pl.BlockSpec((1, dout), lambda i: (0, 0))] * 2
    )
    if split:
        out_shape = [jax.ShapeDtypeStruct((N_PAD, 128), jnp.float32)] * 2
        out_specs = [pl.BlockSpec((BLK, 128), lambda i: (i, 0))] * 2
    else:
        out_shape = [jax.ShapeDtypeStruct((N_PAD, dout), jnp.float32)]
        out_specs = [pl.BlockSpec((BLK, dout), lambda i: (i, 0))]
    if colsum:
        out_shape = out_shape + [jax.ShapeDtypeStruct((1, sum_w), jnp.float32)]
        out_specs = out_specs + [pl.BlockSpec((1, sum_w), lambda i: (0, 0))]

    return pl.pallas_call(
        body,
        grid=(GRID,),
        in_specs=in_specs,
        out_specs=out_specs,
        out_shape=out_shape,
    )


def _dense(parts, W, b, a, row_splits, split_out, colsum=False):
    widths = [int(p.shape[1]) for p in parts]
    ws = [W[s:s + w] for s, w in zip(row_splits, widths)]
    fn = _make_dense(tuple(widths), int(W.shape[1]), split_out, colsum)
    return fn(*parts, *ws, b, a)


def _colsum_pair_body(a0_ref, a1_ref, o0_ref, o1_ref):
    i = pl.program_id(0)

    @pl.when(i == 0)
    def _():
        o0_ref[...] = jnp.zeros_like(o0_ref)
        o1_ref[...] = jnp.zeros_like(o1_ref)

    rows = i * BLK + lax.broadcasted_iota(jnp.int32, (BLK, 1), 0)
    m = (rows < N_NODES).astype(jnp.float32)
    o0_ref[...] += jnp.sum(a0_ref[...] * m, axis=0, keepdims=True)
    o1_ref[...] += jnp.sum(a1_ref[...] * m, axis=0, keepdims=True)


_colsum_pair = pl.pallas_call(
    _colsum_pair_body,
    grid=(GRID,),
    in_specs=[pl.BlockSpec((BLK, 128), lambda i: (i, 0))] * 2,
    out_specs=[pl.BlockSpec((1, 128), lambda i: (0, 0))] * 2,
    out_shape=[jax.ShapeDtypeStruct((1, 128), jnp.float32)] * 2,
)


def _post_body(p_ref, w1_ref, b1_ref, a1_ref, w2t_ref, b2_ref, o_ref):
    y = jnp.dot(p_ref[...], w1_ref[...], preferred_element_type=jnp.float32) + b1_ref[...]
    y = jnp.where(y > 0.0, y, y * a1_ref[...])
    o_ref[...] = jnp.sum(y * w2t_ref[...], axis=1, keepdims=True) + b2_ref[...]


_post = pl.pallas_call(
    _post_body,
    out_shape=jax.ShapeDtypeStruct((1, 1), jnp.float32),
)


def _sc_agg_body(z0, z1, src_hbm, dst_hbm, out0, out1,
                 src_v, dst_v, rows_a, rows_b, agg_sh,
                 gsem_a, gsem_b, ssem_a, ssem_b):
    c = lax.axis_index("c")
    s = lax.axis_index("s")

    zero16 = jnp.zeros((16,), jnp.float32)

    def zb(i, carry):
        for k in range(8):
            rows_a[i, pl.ds(k * 16, 16)] = zero16
        return carry

    lax.fori_loop(0, CHUNK, zb, 0)

    zbase = s * ROWS_PER_TILE
    for r0 in range(0, ROWS_PER_TILE, CHUNK):
        pltpu.sync_copy(rows_a, agg_sh.at[pl.ds(zbase + r0, CHUNK)])
    plsc.subcore_barrier()

    half = STAGE_ROWS // 2

    def run(zt):
        def wait_g(buf, sem):
            pltpu.make_async_copy(zt.at[src_v.at[0]], buf, sem).wait()

        def wait_s(buf, sem):
            pltpu.make_async_copy(buf, agg_sh.at[dst_v.at[0]], sem).wait()

        for st in range(EPW_ROWS // STAGE_ROWS):
            base = s * EPW_ROWS + st * STAGE_ROWS
            pltpu.sync_copy(src_hbm.at[pl.ds(base, STAGE_ROWS)], src_v)
            pltpu.sync_copy(dst_hbm.at[pl.ds(base, STAGE_ROWS)], dst_v)

            pltpu.async_copy(zt.at[src_v.at[0]], rows_a, gsem_a)

            def cb(jj, carry):
                j0 = 2 * jj
                wait_g(rows_a, gsem_a)

                @pl.when(jj > 0)
                def _():
                    wait_s(rows_b, ssem_b)

                pltpu.async_copy(zt.at[src_v.at[j0 + 1]], rows_b, gsem_b)
                pltpu.async_copy(rows_a, agg_sh.at[dst_v.at[j0]], ssem_a, add=True)

                wait_g(rows_b, gsem_b)
                wait_s(rows_a, ssem_a)

                @pl.when(jj < half - 1)
                def _():
                    pltpu.async_copy(zt.at[src_v.at[j0 + 2]], rows_a, gsem_a)

                pltpu.async_copy(rows_b, agg_sh.at[dst_v.at[j0 + 1]], ssem_b, add=True)
                return carry

            lax.fori_loop(0, half, cb, 0)
            wait_s(rows_b, ssem_b)

    @pl.when(c == 0)
    def _():
        run(z0)

    @pl.when(c == 1)
    def _():
        run(z1)

    plsc.subcore_barrier()

    @pl.when(c == 0)
    def _():
        pltpu.sync_copy(agg_sh.at[pl.ds(zbase, ROWS_PER_TILE)],
                        out0.at[pl.ds(zbase, ROWS_PER_TILE)])

    @pl.when(c == 1)
    def _():
        pltpu.sync_copy(agg_sh.at[pl.ds(zbase, ROWS_PER_TILE)],
                        out1.at[pl.ds(zbase, ROWS_PER_TILE)])


def _make_sc_agg():
    return pl.kernel(
        _sc_agg_body,
        out_type=[jax.ShapeDtypeStruct((N_PAD, 128), jnp.float32)] * 2,
        mesh=plsc.VectorSubcoreMesh(core_axis_name="c", subcore_axis_name="s"),
        scratch_types=[
            pltpu.VMEM((STAGE_ROWS, CHUNK), jnp.int32),
            pltpu.VMEM((STAGE_ROWS, CHUNK), jnp.int32),
            pltpu.VMEM((CHUNK, 128), jnp.float32),
            pltpu.VMEM((CHUNK, 128), jnp.float32),
            pltpu.VMEM_SHARED((N_PAD, 128), jnp.float32),
            pltpu.SemaphoreType.DMA,
            pltpu.SemaphoreType.DMA,
            pltpu.SemaphoreType.DMA,
            pltpu.SemaphoreType.DMA,
        ],
    )


def kernel(x, edge_index, params):
    W1, b1, a1 = _fold(params["pre1"])
    W2, b2, a2 = _fold(params["pre2"])
    Wc = [_fold(params["conv%d" % i]) for i in (1, 2, 3, 4)]
    Wp1, bp1, ap1 = _fold(params["post1"])
    Wp2, bp2, _ = _fold(params["post2"])

    xp = jnp.pad(x, ((0, N_PAD - N_NODES), (0, 0)))
    src = edge_index[0]
    dst = edge_index[1]
    pad_e = E_PAD - src.shape[0]
    srcp = jnp.concatenate([src, jnp.zeros((pad_e,), jnp.int32)])
    dstp = jnp.concatenate([dst, jnp.full((pad_e,), N_NODES, jnp.int32)])
    src2d = srcp.reshape(E_PAD // CHUNK, CHUNK)
    dst2d = dstp.reshape(E_PAD // CHUNK, CHUNK)

    sc_agg = _make_sc_agg()

    (t,) = _dense([xp], W1, b1, a1, [0], split_out=False)
    (h,) = _dense([t], W2, b2, a2, [0], split_out=False)

    Wk, bk, ak = Wc[0]
    z0, z1 = _dense([h], Wk, bk, ak, [0], split_out=True)
    g1_0, g1_1 = sc_agg(z0, z1, src2d, dst2d)

    Wk, bk, ak = Wc[1]
    z0, z1 = _dense([g1_0, g1_1, h], Wk, bk, ak, [0, 128, 256], split_out=True)
    g2_0, g2_1 = sc_agg(z0, z1, src2d, dst2d)

    Wk, bk, ak = Wc[2]
    z0, z1 = _dense([g2_0, g2_1, g1_0, g1_1, h], Wk, bk, ak,
                    [0, 128, 256, 384, 512], split_out=True)
    g3_0, g3_1 = sc_agg(z0, z1, src2d, dst2d)

    Wk, bk, ak = Wc[3]
    z0, z1, cs_in = _dense([g3_0, g3_1, g2_0, g2_1, g1_0, g1_1, h], Wk, bk, ak,
                           [0, 128, 256, 384, 512, 640, 768],
                           split_out=True, colsum=True)
    g4_0, g4_1 = sc_agg(z0, z1, src2d, dst2d)
    cs4_0, cs4_1 = _colsum_pair(g4_0, g4_1)

    pooled = jnp.concatenate([cs4_0, cs4_1, cs_in], axis=1)

    y = _post(pooled, Wp1, bp1, ap1, Wp2.T, bp2)
    return y.reshape((1,))
